# baseline (speedup 1.0000x reference)
import functools

import jax
import jax.numpy as jnp
from jax import lax
from jax.experimental import pallas as pl
from jax.experimental.pallas import tpu as pltpu


def _topk_softmax(logits, out_ref, idx_ref, row0, *, k_top, n_e, bn):
    e_iota = lax.broadcasted_iota(
        jnp.int32, (bn, n_e), 1).astype(jnp.float32)
    r_iota = lax.broadcasted_iota(jnp.int32, (bn, k_top), 1)
    work = logits
    sel = jnp.zeros((bn, n_e), jnp.bool_)
    idx_out = jnp.zeros((bn, k_top), jnp.int32)
    top0 = None
    for k in range(k_top):
        m = jnp.max(work, axis=1, keepdims=True)
        hit = work == m
        idxk = jnp.min(jnp.where(hit, e_iota, float(n_e)), axis=1,
                       keepdims=True)
        picked = e_iota == idxk
        work = jnp.where(picked, -jnp.inf, work)
        sel = jnp.logical_or(sel, picked)
        idx_out = jnp.where(r_iota == k, idxk.astype(jnp.int32), idx_out)
        if k == 0:
            top0 = m
    ex = jnp.where(sel, jnp.exp(logits - top0), 0.0)
    out_ref[pl.ds(row0, bn), :] = ex / jnp.sum(ex, axis=1, keepdims=True)
    idx_ref[pl.ds(row0, bn), :] = idx_out


def _router_body(x_ref, w1_ref, b1_ref, w2_ref, b2_ref, out_ref, idx_ref,
                 lg_ref, *, k_top, n_e, bn, prec1, prec2):
    i = pl.program_id(0)
    ni = pl.num_programs(0)

    xb = x_ref[...].astype(jnp.bfloat16)
    hp = lax.dot_general(xb, w1_ref[...], (((1,), (1,)), ((), ())),
                         preferred_element_type=jnp.float32, precision=prec1)
    hb = jnp.maximum(hp + b1_ref[...], 0.0).astype(jnp.bfloat16)
    logits = lax.dot_general(
        hb, w2_ref[...], (((1,), (1,)), ((), ())),
        preferred_element_type=jnp.float32, precision=prec2) + b2_ref[...]
    lg_ref[lax.rem(i, 2)] = logits

    prev = jnp.maximum(i - 1, 0)
    _topk_softmax(lg_ref[lax.rem(prev, 2)], out_ref, idx_ref, prev * bn,
                  k_top=k_top, n_e=n_e, bn=bn)

    @pl.when(i == ni - 1)
    def _flush_last():
        _topk_softmax(lg_ref[lax.rem(i, 2)], out_ref, idx_ref, i * bn,
                      k_top=k_top, n_e=n_e, bn=bn)


@jax.jit
def kernel(x, W1, b1, W2, b2):
    n, d = x.shape
    h_dim = W1.shape[0]
    n_e = W2.shape[0]
    k_top = 8
    bn = min(256, n)
    assert n % bn == 0

    w1b = W1.astype(jnp.bfloat16)
    w2b = W2.astype(jnp.bfloat16)
    b1r = b1.reshape(1, h_dim)
    b2r = b2.reshape(1, n_e)

    body = functools.partial(
        _router_body, k_top=k_top, n_e=n_e, bn=bn,
        prec1=lax.Precision.DEFAULT, prec2=lax.Precision.DEFAULT)

    out, idx = pl.pallas_call(
        body,
        grid=(n // bn,),
        in_specs=[
            pl.BlockSpec((bn, d), lambda i: (i, 0)),
            pl.BlockSpec((h_dim, d), lambda i: (0, 0)),
            pl.BlockSpec((1, h_dim), lambda i: (0, 0)),
            pl.BlockSpec((n_e, h_dim), lambda i: (0, 0)),
            pl.BlockSpec((1, n_e), lambda i: (0, 0)),
        ],
        out_specs=[
            pl.BlockSpec((n, n_e), lambda i: (0, 0)),
            pl.BlockSpec((n, k_top), lambda i: (0, 0)),
        ],
        out_shape=[
            jax.ShapeDtypeStruct((n, n_e), jnp.float32),
            jax.ShapeDtypeStruct((n, k_top), jnp.int32),
        ],
        scratch_shapes=[pltpu.VMEM((2, bn, n_e), jnp.float32)],
        compiler_params=pltpu.CompilerParams(
            dimension_semantics=("arbitrary",)),
    )(x, w1b, b1r, w2b, b2r)
    return (out, idx)


# speedup vs baseline: 1.0759x; 1.0759x over previous
import functools

import jax
import jax.numpy as jnp
from jax import lax
from jax.experimental import pallas as pl
from jax.experimental.pallas import tpu as pltpu


def _router_body(x_ref, w1_ref, b1_ref, w2_ref, b2_ref, out_ref, idx_ref,
                 *, k_top, n_e, bn, prec1, prec2):
    xb = x_ref[...].astype(jnp.bfloat16)
    d = xb.shape[1]
    bk = 512
    hp = None
    for c in range(d // bk):
        part = lax.dot_general(
            xb[:, c * bk:(c + 1) * bk], w1_ref[:, c * bk:(c + 1) * bk],
            (((1,), (1,)), ((), ())),
            preferred_element_type=jnp.float32, precision=prec1)
        hp = part if hp is None else hp + part
    hb = jnp.maximum(hp + b1_ref[...], 0.0).astype(jnp.bfloat16)
    logits = lax.dot_general(
        hb, w2_ref[...], (((1,), (1,)), ((), ())),
        preferred_element_type=jnp.float32, precision=prec2) + b2_ref[...]
    e_iota = lax.broadcasted_iota(
        jnp.int32, (bn, n_e), 1).astype(jnp.float32)
    r_iota = lax.broadcasted_iota(jnp.int32, (bn, k_top), 1)
    work = logits
    sel = jnp.zeros((bn, n_e), jnp.bool_)
    idx_out = jnp.zeros((bn, k_top), jnp.int32)
    top0 = None
    for k in range(k_top):
        m = jnp.max(work, axis=1, keepdims=True)
        hit = work == m
        idxk = jnp.min(jnp.where(hit, e_iota, float(n_e)), axis=1,
                       keepdims=True)
        picked = e_iota == idxk
        work = jnp.where(picked, -jnp.inf, work)
        sel = jnp.logical_or(sel, picked)
        idx_out = jnp.where(r_iota == k, idxk.astype(jnp.int32), idx_out)
        if k == 0:
            top0 = m
    ex = jnp.where(sel, jnp.exp(logits - top0), 0.0)
    out_ref[...] = ex / jnp.sum(ex, axis=1, keepdims=True)
    idx_ref[...] = idx_out


@jax.jit
def kernel(x, W1, b1, W2, b2):
    n, d = x.shape
    h_dim = W1.shape[0]
    n_e = W2.shape[0]
    k_top = 8
    bn = min(512, n)
    assert n % bn == 0

    w1b = W1.astype(jnp.bfloat16)
    w2b = W2.astype(jnp.bfloat16)
    b1r = b1.reshape(1, h_dim)
    b2r = b2.reshape(1, n_e)

    body = functools.partial(
        _router_body, k_top=k_top, n_e=n_e, bn=bn,
        prec1=lax.Precision.DEFAULT, prec2=lax.Precision.DEFAULT)

    out, idx = pl.pallas_call(
        body,
        grid=(n // bn,),
        in_specs=[
            pl.BlockSpec((bn, d), lambda i: (i, 0)),
            pl.BlockSpec((h_dim, d), lambda i: (0, 0)),
            pl.BlockSpec((1, h_dim), lambda i: (0, 0)),
            pl.BlockSpec((n_e, h_dim), lambda i: (0, 0)),
            pl.BlockSpec((1, n_e), lambda i: (0, 0)),
        ],
        out_specs=[
            pl.BlockSpec((bn, n_e), lambda i: (i, 0)),
            pl.BlockSpec((bn, k_top), lambda i: (i, 0)),
        ],
        out_shape=[
            jax.ShapeDtypeStruct((n, n_e), jnp.float32),
            jax.ShapeDtypeStruct((n, k_top), jnp.int32),
        ],
        compiler_params=pltpu.CompilerParams(
            dimension_semantics=("arbitrary",)),
    )(x, w1b, b1r, w2b, b2r)
    return (out, idx)
